# baseline (speedup 1.0000x reference)
import functools

import jax
import jax.numpy as jnp
from jax import lax
from jax.experimental import pallas as pl
from jax.experimental.pallas import tpu as pltpu
from jax.experimental.pallas import tpu_sc as plsc

B = 16384
TAB = 64
EMB = 32
NTAB = 4
EPS = 1e-5
H1, H2, H3 = 256, 128, 64
RAWW = NTAB * 128

_NC = 2
_NS = 16
_NW = _NC * _NS
_BPW = B // _NW
_CHUNK = 128
_NCH = _BPW // _CHUNK


def _sc_gather_body(t0, t1, t2, t3, h0, h1_, h2_, h3_, out,
                    hi_v, blk, sem):
    wid = lax.axis_index("s") * _NC + lax.axis_index("c")
    base = wid * _BPW
    tables = (t0, t1, t2, t3)
    his = (h0, h1_, h2_, h3_)
    for t in range(NTAB):
        pltpu.sync_copy(his[t].at[pl.ds(base, _BPW)], hi_v)
        copies = [
            pltpu.async_copy(
                tables[t].at[hi_v.at[pl.ds(j * _CHUNK, _CHUNK)]],
                blk.at[pl.ds(j * _CHUNK, _CHUNK)], sem)
            for j in range(_NCH)
        ]
        for c in copies:
            c.wait()
        pltpu.sync_copy(
            blk, out.at[pl.ds(base, _BPW), pl.ds(t * 128, 128)])


@functools.lru_cache(maxsize=1)
def _sc_gather():
    return pl.kernel(
        _sc_gather_body,
        out_type=jax.ShapeDtypeStruct((B, RAWW), jnp.int32),
        mesh=plsc.VectorSubcoreMesh(core_axis_name="c", subcore_axis_name="s"),
        scratch_types=[
            pltpu.VMEM((_BPW,), jnp.int32),
            pltpu.VMEM((_BPW, 128), jnp.int32),
            pltpu.SemaphoreType.DMA,
        ],
    )


_R = 1024
_NB = B // _R


def _mlp_body(tab_ref, sel_ref, raw_hbm, w1t_ref, w1e_ref, b1_ref, g1_ref,
              be1_ref, w2_ref, b2_ref, g2_ref, be2_ref,
              w3_ref, b3_ref, g3_ref, be3_ref,
              w4_ref, b4_ref, out_ref, buf, a1_s, a2_s, a3_s, sem):
    f32 = jnp.float32

    def dot(a, b):
        return lax.dot_general(a, b, (((1,), (0,)), ((), ())),
                               preferred_element_type=f32)

    def start_raw(i, slot):
        pltpu.make_async_copy(
            raw_hbm.at[pl.ds(i * _R, _R)], buf.at[slot], sem.at[slot]
        ).start()

    def wait_raw(i, slot):
        pltpu.make_async_copy(
            raw_hbm.at[pl.ds(i * _R, _R)], buf.at[slot], sem.at[slot]
        ).wait()

    w1t = w1t_ref[...]
    w1e = w1e_ref[...]
    b1 = b1_ref[...]

    start_raw(0, 0)

    def s1(i, carry):
        s, ss = carry
        slot = lax.rem(i, 2)

        @pl.when(i + 1 < _NB)
        def _():
            start_raw(i + 1, lax.rem(i + 1, 2))

        wait_raw(i, slot)
        r0 = pl.multiple_of(i * _R, _R)
        raw = buf[slot]
        sel = sel_ref[pl.ds(r0, _R), :]
        parts = []
        for t in range(NTAB):
            st = sel[:, t:t + 1]
            u = jnp.zeros((_R, 16), jnp.int32)
            for g in range(8):
                piece = raw[:, t * 128 + g * 16:t * 128 + (g + 1) * 16]
                u = jnp.where(st == g, piece, u)
            lo = lax.bitcast_convert_type(u << 16, f32)
            hi = lax.bitcast_convert_type(u & jnp.int32(-65536), f32)
            parts.append(lo)
            parts.append(hi)
        x_emb = jnp.concatenate(parts, axis=1)
        a = dot(tab_ref[pl.ds(r0, _R), :], w1t) + dot(x_emb, w1e) + b1
        a1_s[pl.ds(r0, _R), :] = a.astype(jnp.bfloat16)
        return (s + jnp.sum(a, 0, keepdims=True),
                ss + jnp.sum(a * a, 0, keepdims=True))

    z1 = jnp.zeros((1, H1), f32)
    s, ss = lax.fori_loop(0, _NB, s1, (z1, z1))
    mu = s * (1.0 / B)
    var = ss * (1.0 / B) - mu * mu
    al1 = g1_ref[...] * lax.rsqrt(var + EPS)
    bt1 = be1_ref[...] - mu * al1

    w2 = w2_ref[...]
    b2 = b2_ref[...]

    def s2(i, carry):
        s, ss = carry
        r0 = pl.multiple_of(i * _R, _R)
        h = jnp.maximum(
            a1_s[pl.ds(r0, _R), :].astype(jnp.float32) * al1 + bt1, 0.0)
        a = dot(h, w2) + b2
        a2_s[pl.ds(r0, _R), :] = a
        return (s + jnp.sum(a, 0, keepdims=True),
                ss + jnp.sum(a * a, 0, keepdims=True))

    z2 = jnp.zeros((1, H2), f32)
    s, ss = lax.fori_loop(0, _NB, s2, (z2, z2))
    mu = s * (1.0 / B)
    var = ss * (1.0 / B) - mu * mu
    al2 = g2_ref[...] * lax.rsqrt(var + EPS)
    bt2 = be2_ref[...] - mu * al2

    w3 = w3_ref[...]
    b3 = b3_ref[...]

    def s3(i, carry):
        s, ss = carry
        r0 = pl.multiple_of(i * _R, _R)
        h = jnp.maximum(a2_s[pl.ds(r0, _R), :] * al2 + bt2, 0.0)
        a = dot(h, w3) + b3
        a3_s[pl.ds(r0, _R), :] = a
        return (s + jnp.sum(a, 0, keepdims=True),
                ss + jnp.sum(a * a, 0, keepdims=True))

    z3 = jnp.zeros((1, H3), f32)
    s, ss = lax.fori_loop(0, _NB, s3, (z3, z3))
    mu = s * (1.0 / B)
    var = ss * (1.0 / B) - mu * mu
    al3 = g3_ref[...] * lax.rsqrt(var + EPS)
    bt3 = be3_ref[...] - mu * al3

    w4 = w4_ref[...]
    b4s = b4_ref[0]

    def s4(i, _):
        r0 = pl.multiple_of(i * _R, _R)
        h = jnp.maximum(a3_s[pl.ds(r0, _R), :] * al3 + bt3, 0.0)
        out_ref[pl.ds(r0, _R)] = jnp.sum(h * w4, axis=1) + b4s
        return 0

    lax.fori_loop(0, _NB, s4, 0)


def _mlp_call(tabular, sel, raw, w1t, w1e, b1, g1, be1, w2, b2, g2, be2,
              w3, b3, g3, be3, w4, b4):
    in_specs = [pl.BlockSpec(memory_space=pltpu.VMEM)] * 18
    in_specs[2] = pl.BlockSpec(memory_space=pltpu.MemorySpace.HBM)
    in_specs[17] = pl.BlockSpec(memory_space=pltpu.SMEM)
    return pl.pallas_call(
        _mlp_body,
        out_shape=jax.ShapeDtypeStruct((B,), jnp.float32),
        in_specs=in_specs,
        out_specs=pl.BlockSpec(memory_space=pltpu.VMEM),
        scratch_shapes=[
            pltpu.VMEM((2, _R, RAWW), jnp.int32),
            pltpu.VMEM((B, H1), jnp.bfloat16),
            pltpu.VMEM((B, H2), jnp.float32),
            pltpu.VMEM((B, H3), jnp.float32),
            pltpu.SemaphoreType.DMA((2,)),
        ],
    )(tabular, sel, raw, w1t, w1e, b1, g1, be1, w2, b2, g2, be2,
      w3, b3, g3, be3, w4, b4)


def kernel(tabular, song, venue, tour, country,
           song_table, venue_table, tour_table, country_table,
           W1, b1, g1, be1, W2, b2, g2, be2, W3, b3, g3, be3, W4, b4):
    idx = [v.astype(jnp.int32) for v in (song, venue, tour, country)]
    tabs = []
    for tbl in (song_table, venue_table, tour_table, country_table):
        v = tbl.shape[0] - 1
        xb = tbl[:v].astype(jnp.bfloat16).reshape(v // 8, 128, 2)
        tabs.append(lax.bitcast_convert_type(xb, jnp.int32))
    raw = _sc_gather()(*tabs, *[v >> 3 for v in idx])
    sel = jnp.stack([v & 7 for v in idx], axis=1)
    perm = jnp.concatenate([
        t * EMB + jnp.concatenate([jnp.arange(0, EMB, 2),
                                   jnp.arange(1, EMB, 2)])
        for t in range(NTAB)])
    w1e = W1[TAB:][perm]
    r = lambda v: v.reshape(1, -1)
    return _mlp_call(tabular, sel, raw,
                     W1[:TAB], w1e, r(b1), r(g1), r(be1),
                     W2, r(b2), r(g2), r(be2),
                     W3, r(b3), r(g3), r(be3),
                     r(W4[:, 0]), b4)


# speedup vs baseline: 12.6520x; 12.6520x over previous
import functools

import jax
import jax.numpy as jnp
from jax import lax
from jax.experimental import pallas as pl
from jax.experimental.pallas import tpu as pltpu
from jax.experimental.pallas import tpu_sc as plsc

B = 16384
TAB = 64
EMB = 32
NTAB = 4
EPS = 1e-5
H1, H2, H3 = 256, 128, 64
RAWW = NTAB * 128

_NC = 2
_NS = 16
_NW = _NC * _NS
_BPW = B // _NW
_CHUNK = 128
_NCH = _BPW // _CHUNK


def _sc_gather_body(t0, t1, t2, t3, h0, h1_, h2_, h3_, out,
                    hi_v, blk, sem):
    wid = lax.axis_index("s") * _NC + lax.axis_index("c")
    base = wid * _BPW
    tables = (t0, t1, t2, t3)
    his = (h0, h1_, h2_, h3_)
    for t in range(NTAB):
        pltpu.sync_copy(his[t].at[pl.ds(base, _BPW)], hi_v)
        copies = [
            pltpu.async_copy(
                tables[t].at[hi_v.at[pl.ds(j * _CHUNK, _CHUNK)]],
                blk.at[pl.ds(j * _CHUNK, _CHUNK)], sem)
            for j in range(_NCH)
        ]
        for c in copies:
            c.wait()
        pltpu.sync_copy(
            blk, out.at[pl.ds(base, _BPW), pl.ds(t * 128, 128)])


@functools.lru_cache(maxsize=1)
def _sc_gather():
    return pl.kernel(
        _sc_gather_body,
        out_type=jax.ShapeDtypeStruct((B, RAWW), jnp.float32),
        mesh=plsc.VectorSubcoreMesh(core_axis_name="c", subcore_axis_name="s"),
        scratch_types=[
            pltpu.VMEM((_BPW,), jnp.int32),
            pltpu.VMEM((_BPW, 128), jnp.float32),
            pltpu.SemaphoreType.DMA,
        ],
    )


_R = 1024
_NB = B // _R


def _mlp_body(tab_ref, sel_ref, raw_hbm, w1t_ref, w1e_ref, b1_ref, g1_ref,
              be1_ref, w2_ref, b2_ref, g2_ref, be2_ref,
              w3_ref, b3_ref, g3_ref, be3_ref,
              w4_ref, b4_ref, out_ref, buf, a1_s, a2_s, a3_s, sem):
    f32 = jnp.float32

    def dot(a, b):
        return lax.dot_general(a, b, (((1,), (0,)), ((), ())),
                               preferred_element_type=f32)

    def start_raw(i, slot):
        pltpu.make_async_copy(
            raw_hbm.at[pl.ds(i * _R, _R)], buf.at[slot], sem.at[slot]
        ).start()

    def wait_raw(i, slot):
        pltpu.make_async_copy(
            raw_hbm.at[pl.ds(i * _R, _R)], buf.at[slot], sem.at[slot]
        ).wait()

    w1t = w1t_ref[...]
    w1e = w1e_ref[...]
    b1 = b1_ref[...]

    start_raw(0, 0)

    def s1(i, carry):
        s, ss = carry
        slot = lax.rem(i, 2)

        @pl.when(i + 1 < _NB)
        def _():
            start_raw(i + 1, lax.rem(i + 1, 2))

        wait_raw(i, slot)
        r0 = pl.multiple_of(i * _R, _R)
        raw = buf[slot]
        sel = sel_ref[pl.ds(r0, _R), :]
        parts = []
        for t in range(NTAB):
            st = sel[:, t:t + 1]
            acc = jnp.zeros((_R, EMB), f32)
            for g in range(4):
                piece = raw[:, t * 128 + g * EMB:t * 128 + (g + 1) * EMB]
                acc = acc + jnp.where(st == g, piece, 0.0)
            parts.append(acc)
        x_emb = jnp.concatenate(parts, axis=1)
        a = dot(tab_ref[pl.ds(r0, _R), :], w1t) + dot(x_emb, w1e) + b1
        a1_s[pl.ds(r0, _R), :] = a.astype(jnp.bfloat16)
        return (s + jnp.sum(a, 0, keepdims=True),
                ss + jnp.sum(a * a, 0, keepdims=True))

    z1 = jnp.zeros((1, H1), f32)
    s, ss = lax.fori_loop(0, _NB, s1, (z1, z1))
    mu = s * (1.0 / B)
    var = ss * (1.0 / B) - mu * mu
    al1 = g1_ref[...] * lax.rsqrt(var + EPS)
    bt1 = be1_ref[...] - mu * al1

    w2 = w2_ref[...]
    b2 = b2_ref[...]

    def s2(i, carry):
        s, ss = carry
        r0 = pl.multiple_of(i * _R, _R)
        h = jnp.maximum(
            a1_s[pl.ds(r0, _R), :].astype(jnp.float32) * al1 + bt1, 0.0)
        a = dot(h, w2) + b2
        a2_s[pl.ds(r0, _R), :] = a
        return (s + jnp.sum(a, 0, keepdims=True),
                ss + jnp.sum(a * a, 0, keepdims=True))

    z2 = jnp.zeros((1, H2), f32)
    s, ss = lax.fori_loop(0, _NB, s2, (z2, z2))
    mu = s * (1.0 / B)
    var = ss * (1.0 / B) - mu * mu
    al2 = g2_ref[...] * lax.rsqrt(var + EPS)
    bt2 = be2_ref[...] - mu * al2

    w3 = w3_ref[...]
    b3 = b3_ref[...]

    def s3(i, carry):
        s, ss = carry
        r0 = pl.multiple_of(i * _R, _R)
        h = jnp.maximum(a2_s[pl.ds(r0, _R), :] * al2 + bt2, 0.0)
        a = dot(h, w3) + b3
        a3_s[pl.ds(r0, _R), :] = a
        return (s + jnp.sum(a, 0, keepdims=True),
                ss + jnp.sum(a * a, 0, keepdims=True))

    z3 = jnp.zeros((1, H3), f32)
    s, ss = lax.fori_loop(0, _NB, s3, (z3, z3))
    mu = s * (1.0 / B)
    var = ss * (1.0 / B) - mu * mu
    al3 = g3_ref[...] * lax.rsqrt(var + EPS)
    bt3 = be3_ref[...] - mu * al3

    w4 = w4_ref[...]
    b4s = b4_ref[0]

    def s4(i, _):
        r0 = pl.multiple_of(i * _R, _R)
        h = jnp.maximum(a3_s[pl.ds(r0, _R), :] * al3 + bt3, 0.0)
        out_ref[pl.ds(r0, _R)] = jnp.sum(h * w4, axis=1) + b4s
        return 0

    lax.fori_loop(0, _NB, s4, 0)


def _mlp_call(tabular, sel, raw, w1t, w1e, b1, g1, be1, w2, b2, g2, be2,
              w3, b3, g3, be3, w4, b4):
    in_specs = [pl.BlockSpec(memory_space=pltpu.VMEM)] * 18
    in_specs[2] = pl.BlockSpec(memory_space=pltpu.MemorySpace.HBM)
    in_specs[17] = pl.BlockSpec(memory_space=pltpu.SMEM)
    return pl.pallas_call(
        _mlp_body,
        out_shape=jax.ShapeDtypeStruct((B,), jnp.float32),
        in_specs=in_specs,
        out_specs=pl.BlockSpec(memory_space=pltpu.VMEM),
        scratch_shapes=[
            pltpu.VMEM((2, _R, RAWW), jnp.float32),
            pltpu.VMEM((B, H1), jnp.bfloat16),
            pltpu.VMEM((B, H2), jnp.float32),
            pltpu.VMEM((B, H3), jnp.float32),
            pltpu.SemaphoreType.DMA((2,)),
        ],
    )(tabular, sel, raw, w1t, w1e, b1, g1, be1, w2, b2, g2, be2,
      w3, b3, g3, be3, w4, b4)


def kernel(tabular, song, venue, tour, country,
           song_table, venue_table, tour_table, country_table,
           W1, b1, g1, be1, W2, b2, g2, be2, W3, b3, g3, be3, W4, b4):
    idx = [v.astype(jnp.int32) for v in (song, venue, tour, country)]
    tabs = []
    for tbl in (song_table, venue_table, tour_table, country_table):
        v = tbl.shape[0] - 1
        tabs.append(tbl[:v].reshape(v // 4, 4 * EMB))
    raw = _sc_gather()(*tabs, *[v >> 2 for v in idx])
    sel = jnp.stack([v & 3 for v in idx], axis=1)
    r = lambda v: v.reshape(1, -1)
    return _mlp_call(tabular, sel, raw,
                     W1[:TAB], W1[TAB:], r(b1), r(g1), r(be1),
                     W2, r(b2), r(g2), r(be2),
                     W3, r(b3), r(g3), r(be3),
                     r(W4[:, 0]), b4)
